# baseline (speedup 1.0000x reference)
import functools

import jax
import jax.numpy as jnp
from jax import lax
from jax.experimental import pallas as pl
from jax.experimental.pallas import tpu as pltpu
from jax.experimental.pallas import tpu_sc as plsc

SIGMA = 9.0
NC = 2
NS = 16
NW = NC * NS
L = 16

CG = 625


def _make_sc_partials(n_anchors: int, interpret: bool = False):
    assert n_anchors % (16 * CG) == 0
    n_groups = n_anchors // 16
    n_chunks = n_groups // CG
    in_chunk = CG * 32
    tg_chunk = CG * 48

    mesh = plsc.VectorSubcoreMesh(
        core_axis_name="c", subcore_axis_name="s", num_cores=NC,
        num_subcores=NS)

    @functools.partial(
        pl.kernel,
        mesh=mesh,
        out_type=jax.ShapeDtypeStruct((NW, 2, L), jnp.float32),
        scratch_types=[
            pltpu.VMEM((in_chunk,), jnp.float32),
            pltpu.VMEM((tg_chunk,), jnp.float32),
            pltpu.VMEM((2, L), jnp.float32),
        ],
        interpret=interpret,
        compiler_params=pltpu.CompilerParams(needs_layout_passes=False),
    )
    def sc_partials(inp_hbm, tgt_hbm, out_hbm, in_v, tg_v, out_v):
        wid = lax.axis_index("s") * NC + lax.axis_index("c")
        start_c = (wid * n_chunks) // NW
        end_c = ((wid + 1) * n_chunks) // NW

        lane = lax.iota(jnp.int32, L)
        idx_cls = lane * 3
        idx_r0 = idx_cls + 1
        idx_r1 = idx_cls + 2
        idx_p0 = lane * 2
        idx_p1 = idx_p0 + 1

        inv_sigma = jnp.float32(1.0 / SIGMA)
        half_sigma = jnp.float32(0.5 * SIGMA)
        zero = jnp.zeros((L,), jnp.float32)

        def chunk_body(c, carry):
            acc, cnt = carry
            pltpu.sync_copy(inp_hbm.at[pl.ds(c * in_chunk, in_chunk)], in_v)
            pltpu.sync_copy(tgt_hbm.at[pl.ds(c * tg_chunk, tg_chunk)], tg_v)

            def g_body(j, carry2):
                acc, cnt = carry2
                tb = j * 48
                ib = j * 32
                cls = plsc.load_gather(tg_v, [idx_cls + tb])
                r0 = plsc.load_gather(tg_v, [idx_r0 + tb])
                r1 = plsc.load_gather(tg_v, [idx_r1 + tb])
                p0 = plsc.load_gather(in_v, [idx_p0 + ib])
                p1 = plsc.load_gather(in_v, [idx_p1 + ib])
                d0 = jnp.abs(r0 - p0)
                d1 = jnp.abs(r1 - p1)
                t0 = jnp.minimum(d0, inv_sigma)
                t1 = jnp.minimum(d1, inv_sigma)
                l = (d0 + d1) - (t0 + t1) + half_sigma * (t0 * t0 + t1 * t1)
                m = cls == jnp.float32(1.0)
                acc = acc + jnp.where(m, l, zero)
                cnt = cnt + jnp.where(m, jnp.float32(1.0), zero)
                return acc, cnt

            return lax.fori_loop(0, CG, g_body, (acc, cnt))

        acc, cnt = lax.fori_loop(start_c, end_c, chunk_body, (zero, zero))
        out_v[0, :] = acc
        out_v[1, :] = cnt
        pltpu.sync_copy(out_v, out_hbm.at[wid])

    return sc_partials


def kernel(input, target):
    n = input.shape[1]
    inp_flat = input.reshape(n * 2)
    tgt_flat = target.reshape(n * 3)
    partials = _make_sc_partials(n)(inp_flat, tgt_flat)
    s = jnp.sum(partials[:, 0, :])
    c = jnp.sum(partials[:, 1, :])
    return jnp.where(c > 0, s / jnp.maximum(c, 1.0), jnp.float32(0.0))


# speedup vs baseline: 131.5059x; 131.5059x over previous
import functools

import jax
import jax.numpy as jnp
from jax import lax
from jax.experimental import pallas as pl
from jax.experimental.pallas import tpu as pltpu
from jax.experimental.pallas import tpu_sc as plsc

SIGMA = 9.0
NC = 2
NS = 16
NW = NC * NS
L = 16
BLK = 128

CA = 6400
NB = CA // BLK


def _make_sc_partials(n_anchors: int, interpret: bool = False):
    assert n_anchors % CA == 0
    n_chunks = n_anchors // CA
    groups = CA // L

    mesh = plsc.VectorSubcoreMesh(
        core_axis_name="c", subcore_axis_name="s", num_cores=NC,
        num_subcores=NS)

    @functools.partial(
        pl.kernel,
        mesh=mesh,
        out_type=jax.ShapeDtypeStruct((NW, 2, L), jnp.float32),
        scratch_types=[
            pltpu.VMEM((NB, 2, BLK), jnp.float32),
            pltpu.VMEM((CA,), jnp.float32),
            pltpu.VMEM((CA,), jnp.float32),
            pltpu.VMEM((CA,), jnp.float32),
            pltpu.VMEM((2, L), jnp.float32),
        ],
        interpret=interpret,
        compiler_params=pltpu.CompilerParams(
            needs_layout_passes=False, use_tc_tiling_on_sc=True),
    )
    def sc_partials(inp_hbm, tgt_hbm, out_hbm, in_v, cls_v, r0_v, r1_v,
                    out_v):
        wid = lax.axis_index("s") * NC + lax.axis_index("c")
        start_c = (wid * n_chunks) // NW
        end_c = ((wid + 1) * n_chunks) // NW

        inv_sigma = jnp.float32(1.0 / SIGMA)
        half_sigma = jnp.float32(0.5 * SIGMA)
        zero = jnp.zeros((L,), jnp.float32)

        def chunk_body(c, carry):
            acc, cnt = carry
            a0 = c * CA
            pltpu.sync_copy(inp_hbm.at[pl.ds(c * NB, NB), :, :], in_v)
            pltpu.sync_copy(tgt_hbm.at[0, 0, pl.ds(a0, CA)], cls_v)
            pltpu.sync_copy(tgt_hbm.at[1, 0, pl.ds(a0, CA)], r0_v)
            pltpu.sync_copy(tgt_hbm.at[2, 0, pl.ds(a0, CA)], r1_v)

            def g_body(j, carry2):
                acc, cnt = carry2
                t_off = j * L
                b = j // 8
                sub = (j % 8) * L
                cls = cls_v[pl.ds(t_off, L)]
                r0 = r0_v[pl.ds(t_off, L)]
                r1 = r1_v[pl.ds(t_off, L)]
                p0 = in_v[b, 0, pl.ds(sub, L)]
                p1 = in_v[b, 1, pl.ds(sub, L)]
                d0 = jnp.abs(r0 - p0)
                d1 = jnp.abs(r1 - p1)
                t0 = jnp.minimum(d0, inv_sigma)
                t1 = jnp.minimum(d1, inv_sigma)
                l = (d0 + d1) - (t0 + t1) + half_sigma * (t0 * t0 + t1 * t1)
                m = cls == jnp.float32(1.0)
                acc = acc + jnp.where(m, l, zero)
                cnt = cnt + jnp.where(m, jnp.float32(1.0), zero)
                return acc, cnt

            return lax.fori_loop(0, groups, g_body, (acc, cnt))

        acc, cnt = lax.fori_loop(start_c, end_c, chunk_body, (zero, zero))
        out_v[0, :] = acc
        out_v[1, :] = cnt
        pltpu.sync_copy(out_v, out_hbm.at[wid])

    return sc_partials


def kernel(input, target):
    n = input.shape[1]
    tgt_pl = jnp.transpose(target, (2, 0, 1))
    inp_pl = input.reshape(n // BLK, BLK, 2).transpose(0, 2, 1)
    partials = _make_sc_partials(n)(inp_pl, tgt_pl)
    s = jnp.sum(partials[:, 0, :])
    c = jnp.sum(partials[:, 1, :])
    return jnp.where(c > 0, s / jnp.maximum(c, 1.0), jnp.float32(0.0))


# speedup vs baseline: 227.3381x; 1.7287x over previous
import functools

import jax
import jax.numpy as jnp
from jax import lax
from jax.experimental import pallas as pl
from jax.experimental.pallas import tpu as pltpu
from jax.experimental.pallas import tpu_sc as plsc

SIGMA = 9.0
NC = 2
NS = 16
NW = NC * NS
L = 16
BLK = 128
U = BLK // L

CA = 6400
NB = CA // BLK


def _make_sc_partials(n_anchors: int, interpret: bool = False):
    assert n_anchors % CA == 0
    n_chunks = n_anchors // CA

    mesh = plsc.VectorSubcoreMesh(
        core_axis_name="c", subcore_axis_name="s", num_cores=NC,
        num_subcores=NS)

    @functools.partial(
        pl.kernel,
        mesh=mesh,
        out_type=jax.ShapeDtypeStruct((NW, 2, L), jnp.float32),
        scratch_types=[
            pltpu.VMEM((2, NB, 2, BLK), jnp.float32),
            pltpu.VMEM((2, CA), jnp.float32),
            pltpu.VMEM((2, CA), jnp.float32),
            pltpu.VMEM((2, CA), jnp.float32),
            pltpu.VMEM((2, L), jnp.float32),
            pltpu.SemaphoreType.DMA((2,)),
        ],
        interpret=interpret,
        compiler_params=pltpu.CompilerParams(
            needs_layout_passes=False, use_tc_tiling_on_sc=True),
    )
    def sc_partials(inp_hbm, tgt_hbm, out_hbm, in_v, cls_v, r0_v, r1_v,
                    out_v, sem):
        wid = lax.axis_index("s") * NC + lax.axis_index("c")
        start_c = (wid * n_chunks) // NW
        end_c = ((wid + 1) * n_chunks) // NW

        inv_sigma = jnp.float32(1.0 / SIGMA)
        half_sigma = jnp.float32(0.5 * SIGMA)
        one = jnp.float32(1.0)
        zero = jnp.zeros((L,), jnp.float32)

        def chunk_copies(c):
            d = lax.rem(c, 2)
            return (
                pltpu.make_async_copy(
                    inp_hbm.at[pl.ds(c * NB, NB), :, :], in_v.at[d],
                    sem.at[d]),
                pltpu.make_async_copy(
                    tgt_hbm.at[0, 0, pl.ds(c * CA, CA)], cls_v.at[d],
                    sem.at[d]),
                pltpu.make_async_copy(
                    tgt_hbm.at[1, 0, pl.ds(c * CA, CA)], r0_v.at[d],
                    sem.at[d]),
                pltpu.make_async_copy(
                    tgt_hbm.at[2, 0, pl.ds(c * CA, CA)], r1_v.at[d],
                    sem.at[d]),
            )

        def start_chunk(c):
            for cp in chunk_copies(c):
                cp.start()

        def wait_chunk(c):
            for cp in chunk_copies(c):
                cp.wait()

        start_chunk(start_c)

        def chunk_body(c, carry):
            @pl.when(c + 1 < end_c)
            def _():
                start_chunk(c + 1)

            wait_chunk(c)
            d = lax.rem(c, 2)

            def blk_body(g, carry2):
                accs = list(carry2[:U])
                cnts = list(carry2[U:])
                for u in range(U):
                    cls = cls_v[d, pl.ds(g * BLK + u * L, L)]
                    r0 = r0_v[d, pl.ds(g * BLK + u * L, L)]
                    r1 = r1_v[d, pl.ds(g * BLK + u * L, L)]
                    p0 = in_v[d, g, 0, pl.ds(u * L, L)]
                    p1 = in_v[d, g, 1, pl.ds(u * L, L)]
                    d0 = jnp.abs(r0 - p0)
                    d1 = jnp.abs(r1 - p1)
                    t0 = jnp.minimum(d0, inv_sigma)
                    t1 = jnp.minimum(d1, inv_sigma)
                    w = ((d0 - t0) + (d1 - t1)
                         + half_sigma * (t0 * t0 + t1 * t1))
                    mf = jnp.where(cls == one, one, jnp.float32(0.0))
                    accs[u] = accs[u] + w * mf
                    cnts[u] = cnts[u] + mf
                return tuple(accs) + tuple(cnts)

            return lax.fori_loop(0, NB, blk_body, carry)

        init = (zero,) * (2 * U)
        fin = lax.fori_loop(start_c, end_c, chunk_body, init)
        acc = fin[0]
        cnt = fin[U]
        for u in range(1, U):
            acc = acc + fin[u]
            cnt = cnt + fin[U + u]
        out_v[0, :] = acc
        out_v[1, :] = cnt
        pltpu.sync_copy(out_v, out_hbm.at[wid])

    return sc_partials


def kernel(input, target):
    n = input.shape[1]
    tgt_pl = jnp.transpose(target, (2, 0, 1))
    inp_pl = input.reshape(n // BLK, BLK, 2).transpose(0, 2, 1)
    partials = _make_sc_partials(n)(inp_pl, tgt_pl)
    s = jnp.sum(partials[:, 0, :])
    c = jnp.sum(partials[:, 1, :])
    return jnp.where(c > 0, s / jnp.maximum(c, 1.0), jnp.float32(0.0))
